# baseline (speedup 1.0000x reference)
import functools

import jax
import jax.numpy as jnp
from jax import lax
from jax.experimental import pallas as pl
from jax.experimental.pallas import tpu as pltpu
from jax.experimental.pallas import tpu_sc as plsc

T, D, E, F, K = 2048, 1024, 8, 4096, 2
BT = 128
FB = 1024
NF = F // FB
NB = 40
NBPAD = 48
P_PAD = NB * BT
NW = 32
TPW = T // NW
CH = 32
NCH = TPW // CH


def _router_body(x_ref, rw_ref, rb_ref, w0_ref, w1_ref, p0_ref, p1_ref,
                 be_ref, nv_ref):
    logits = jnp.dot(x_ref[...], rw_ref[...],
                     preferred_element_type=jnp.float32) + rb_ref[...]
    col = lax.broadcasted_iota(jnp.int32, (T, E), 1)
    m1 = jnp.max(logits, axis=1, keepdims=True)
    i1 = jnp.min(jnp.where(logits == m1, col, E), axis=1, keepdims=True)
    l2 = jnp.where(col == i1, -jnp.inf, logits)
    m2 = jnp.max(l2, axis=1, keepdims=True)
    i2 = jnp.min(jnp.where(l2 == m2, col, E), axis=1, keepdims=True)
    ev = jnp.exp(m2 - m1)
    ones16 = jnp.ones((1, 128), jnp.float32)
    w0_ref[...] = (1.0 / (1.0 + ev)) * ones16
    w1_ref[...] = (ev / (1.0 + ev)) * ones16

    m0 = (col == i1).astype(jnp.float32)
    m1h = (col == i2).astype(jnp.float32)

    def cumsum_t(m):
        c = m
        s = 1
        while s < T:
            c = c + jnp.concatenate(
                [jnp.zeros((s, E), jnp.float32), c[:T - s]], axis=0)
            s *= 2
        return c
    c0 = cumsum_t(m0)
    c1 = cumsum_t(m1h)
    counts0 = c0[T - 1:T, :]
    counts = counts0 + c1[T - 1:T, :]

    rank0 = jnp.sum((c0 - 1.0) * m0, axis=1, keepdims=True)
    rank1 = jnp.sum(((c1 - 1.0) + counts0) * m1h, axis=1, keepdims=True)

    nb = jnp.floor((counts + (BT - 1)) * (1.0 / BT))
    r8 = lax.broadcasted_iota(jnp.int32, (E, E), 0)
    c8 = lax.broadcasted_iota(jnp.int32, (E, E), 1)
    u8 = (r8 <= c8).astype(jnp.float32)
    incl_nb = jnp.dot(nb, u8, preferred_element_type=jnp.float32)
    excl_nb = incl_nb - nb
    po = excl_nb * BT

    p0_ref[...] = (rank0 + jnp.sum(po * m0, axis=1, keepdims=True)
                   ).astype(jnp.int32)
    p1_ref[...] = (rank1 + jnp.sum(po * m1h, axis=1, keepdims=True)
                   ).astype(jnp.int32)

    bcol = lax.broadcasted_iota(jnp.int32, (NBPAD, 1), 0)
    inbi = incl_nb.astype(jnp.int32)
    be = jnp.sum((bcol >= inbi).astype(jnp.int32), axis=1, keepdims=True)
    be_ref[...] = jnp.minimum(be, E - 1)
    nv = jnp.sum(nb).astype(jnp.int32)
    nv_ref[...] = jnp.reshape(nv, (1, 1))


def _router_call(x, router_w, router_b):
    return pl.pallas_call(
        _router_body,
        out_shape=(
            jax.ShapeDtypeStruct((T, 128), jnp.float32),
            jax.ShapeDtypeStruct((T, 128), jnp.float32),
            jax.ShapeDtypeStruct((T, 1), jnp.int32),
            jax.ShapeDtypeStruct((T, 1), jnp.int32),
            jax.ShapeDtypeStruct((NBPAD, 1), jnp.int32),
            jax.ShapeDtypeStruct((1, 1), jnp.int32),
        ),
    )(x, router_w, router_b.reshape(1, E))


def _dispatch_body(x_hbm, p0_hbm, p1_hbm, w0_hbm, w1_hbm, xs_hbm, wsl_hbm,
                   idx0_v, idx1_v, wrow0, wrow1, xbuf,
                   sem0, sem1, sem2, sem3):
    wid = lax.axis_index("s") * 2 + lax.axis_index("c")
    pltpu.sync_copy(p0_hbm.at[wid], idx0_v)
    pltpu.sync_copy(p1_hbm.at[wid], idx1_v)
    for c in range(NCH):
        base = wid * TPW + c * CH
        pltpu.sync_copy(w0_hbm.at[pl.ds(base, CH)], wrow0)
        pltpu.sync_copy(w1_hbm.at[pl.ds(base, CH)], wrow1)
        pltpu.sync_copy(x_hbm.at[pl.ds(base, CH)], xbuf)
        cp0 = pltpu.async_copy(xbuf, xs_hbm.at[idx0_v.at[c]], sem0)
        cp1 = pltpu.async_copy(xbuf, xs_hbm.at[idx1_v.at[c]], sem1)
        cp2 = pltpu.async_copy(wrow0, wsl_hbm.at[idx0_v.at[c]], sem2)
        cp3 = pltpu.async_copy(wrow1, wsl_hbm.at[idx1_v.at[c]], sem3)
        cp0.wait()
        cp1.wait()
        cp2.wait()
        cp3.wait()


def _dispatch_call(x, p0r, p1r, w0r, w1r):
    return pl.kernel(
        _dispatch_body,
        out_type=(jax.ShapeDtypeStruct((P_PAD, D), jnp.float32),
                  jax.ShapeDtypeStruct((P_PAD, 128), jnp.float32)),
        mesh=plsc.VectorSubcoreMesh(core_axis_name="c", subcore_axis_name="s"),
        scratch_types=[
            pltpu.VMEM((NCH, CH), jnp.int32),
            pltpu.VMEM((NCH, CH), jnp.int32),
            pltpu.VMEM((CH, 128), jnp.float32),
            pltpu.VMEM((CH, 128), jnp.float32),
            pltpu.VMEM((CH, D), jnp.float32),
            pltpu.SemaphoreType.DMA,
            pltpu.SemaphoreType.DMA,
            pltpu.SemaphoreType.DMA,
            pltpu.SemaphoreType.DMA,
        ],
    )(x, p0r, p1r, w0r, w1r)


def _ffn_body(be_ref, nv_ref, xs_ref, w1_ref, b1_ref, w2_ref, b2_ref,
              wsl_ref, y_ref, acc_ref):
    f = pl.program_id(0)
    b = pl.program_id(1)

    @pl.when(b < nv_ref[0])
    def _():
        h = jnp.maximum(
            jnp.dot(xs_ref[...], w1_ref[0],
                    preferred_element_type=jnp.float32) + b1_ref[0], 0.0)
        part = jnp.dot(h, w2_ref[0], preferred_element_type=jnp.float32)

        @pl.when(f == 0)
        def _():
            acc_ref[b] = part

        @pl.when(jnp.logical_and(f > 0, f < NF - 1))
        def _():
            acc_ref[b] += part

        @pl.when(f == NF - 1)
        def _():
            y_ref[...] = (acc_ref[b] + part + b2_ref[0]) * wsl_ref[:, 0:1]


def _ffn_call(be32, nv1, xs, wsl, w1, b1, w2, b2):
    def eclamp(f, b, be, nv):
        return be[jnp.where(b < nv[0], b, nv[0] - 1)]

    grid_spec = pltpu.PrefetchScalarGridSpec(
        num_scalar_prefetch=2,
        grid=(NF, NB),
        in_specs=[
            pl.BlockSpec((BT, D),
                         lambda f, b, be, nv: (jnp.minimum(b, nv[0] - 1), 0)),
            pl.BlockSpec((1, D, FB),
                         lambda f, b, be, nv: (eclamp(f, b, be, nv), 0, f)),
            pl.BlockSpec((1, 1, FB),
                         lambda f, b, be, nv: (eclamp(f, b, be, nv), 0, f)),
            pl.BlockSpec((1, FB, D),
                         lambda f, b, be, nv: (eclamp(f, b, be, nv), f, 0)),
            pl.BlockSpec((1, 1, D),
                         lambda f, b, be, nv: (eclamp(f, b, be, nv), 0, 0)),
            pl.BlockSpec((BT, 128),
                         lambda f, b, be, nv: (jnp.minimum(b, nv[0] - 1), 0)),
        ],
        out_specs=pl.BlockSpec(
            (BT, D), lambda f, b, be, nv: (jnp.where(f == NF - 1, b, NB), 0)),
        scratch_shapes=[pltpu.VMEM((NB, BT, D), jnp.float32)],
    )
    return pl.pallas_call(
        _ffn_body,
        grid_spec=grid_spec,
        out_shape=jax.ShapeDtypeStruct((P_PAD + BT, D), jnp.float32),
        compiler_params=pltpu.CompilerParams(
            dimension_semantics=("arbitrary", "arbitrary")),
    )(be32, nv1, xs, w1, b1.reshape(E, 1, F), w2, b2.reshape(E, 1, D), wsl)


CHG = 16
NCHG = TPW // CHG


def _gather_body(y_hbm, p0_hbm, p1_hbm, o_hbm, idx0_v, idx1_v,
                 buf0a, buf1a, buf0b, buf1b, sem0a, sem1a, sem0b, sem1b):
    wid = lax.axis_index("s") * 2 + lax.axis_index("c")
    pltpu.sync_copy(p0_hbm.at[wid], idx0_v)
    pltpu.sync_copy(p1_hbm.at[wid], idx1_v)
    bufs = [(buf0a, buf1a, sem0a, sem1a), (buf0b, buf1b, sem0b, sem1b)]

    def start(c):
        b0, b1, s0, s1 = bufs[c % 2]
        cp0 = pltpu.async_copy(y_hbm.at[idx0_v.at[c]], b0, s0)
        cp1 = pltpu.async_copy(y_hbm.at[idx1_v.at[c]], b1, s1)
        return cp0, cp1

    inflight = start(0)
    for c in range(NCHG):
        b0, b1, _, _ = bufs[c % 2]
        cp0, cp1 = inflight
        cp0.wait()
        cp1.wait()
        if c + 1 < NCHG:
            inflight = start(c + 1)

        def row(i, carry):
            for j in range(D // 16):
                sl = pl.ds(j * 16, 16)
                b0[i, sl] = b0[i, sl] + b1[i, sl]
            return carry

        lax.fori_loop(0, CHG, row, 0)
        pltpu.sync_copy(b0, o_hbm.at[pl.ds(wid * TPW + c * CHG, CHG)])


def _gather_call(y, p0g, p1g):
    return pl.kernel(
        _gather_body,
        out_type=jax.ShapeDtypeStruct((T, D), jnp.float32),
        mesh=plsc.VectorSubcoreMesh(core_axis_name="c", subcore_axis_name="s"),
        scratch_types=[
            pltpu.VMEM((NCHG, CHG), jnp.int32),
            pltpu.VMEM((NCHG, CHG), jnp.int32),
            pltpu.VMEM((CHG, D), jnp.float32),
            pltpu.VMEM((CHG, D), jnp.float32),
            pltpu.VMEM((CHG, D), jnp.float32),
            pltpu.VMEM((CHG, D), jnp.float32),
            pltpu.SemaphoreType.DMA,
            pltpu.SemaphoreType.DMA,
            pltpu.SemaphoreType.DMA,
            pltpu.SemaphoreType.DMA,
        ],
    )(y, p0g, p1g)


@jax.jit
def kernel(x, router_w, router_b, w1, b1, w2, b2):
    w0c, w1c, p0, p1, be, nv = _router_call(x, router_w, router_b)
    p0r = p0.reshape(NW, NCH, CH)
    p1r = p1.reshape(NW, NCH, CH)
    xs, wsl = _dispatch_call(x, p0r, p1r, w0c, w1c)
    y = _ffn_call(be.reshape(NBPAD)[:NB], nv.reshape(1), xs, wsl,
                  w1, b1, w2, b2)
    return _gather_call(y, p0.reshape(NW, NCHG, CHG), p1.reshape(NW, NCHG, CHG))


# speedup vs baseline: 1.0173x; 1.0173x over previous
import functools

import jax
import jax.numpy as jnp
from jax import lax
from jax.experimental import pallas as pl
from jax.experimental.pallas import tpu as pltpu
from jax.experimental.pallas import tpu_sc as plsc

T, D, E, F, K = 2048, 1024, 8, 4096, 2
BT = 256
FB = 1024
NF = F // FB
NB = 24
NBPAD = 32
P_PAD = NB * BT
NW = 32
TPW = T // NW
CH = 32
NCH = TPW // CH


def _router_body(x_ref, rw_ref, rb_ref, w0_ref, w1_ref, p0_ref, p1_ref,
                 be_ref, nv_ref):
    logits = jnp.dot(x_ref[...], rw_ref[...],
                     preferred_element_type=jnp.float32) + rb_ref[...]
    col = lax.broadcasted_iota(jnp.int32, (T, E), 1)
    m1 = jnp.max(logits, axis=1, keepdims=True)
    i1 = jnp.min(jnp.where(logits == m1, col, E), axis=1, keepdims=True)
    l2 = jnp.where(col == i1, -jnp.inf, logits)
    m2 = jnp.max(l2, axis=1, keepdims=True)
    i2 = jnp.min(jnp.where(l2 == m2, col, E), axis=1, keepdims=True)
    ev = jnp.exp(m2 - m1)
    ones16 = jnp.ones((1, 128), jnp.float32)
    w0_ref[...] = (1.0 / (1.0 + ev)) * ones16
    w1_ref[...] = (ev / (1.0 + ev)) * ones16

    m0 = (col == i1).astype(jnp.float32)
    m1h = (col == i2).astype(jnp.float32)

    def cumsum_t(m):
        c = m
        s = 1
        while s < T:
            c = c + jnp.concatenate(
                [jnp.zeros((s, E), jnp.float32), c[:T - s]], axis=0)
            s *= 2
        return c
    c0 = cumsum_t(m0)
    c1 = cumsum_t(m1h)
    counts0 = c0[T - 1:T, :]
    counts = counts0 + c1[T - 1:T, :]

    rank0 = jnp.sum((c0 - 1.0) * m0, axis=1, keepdims=True)
    rank1 = jnp.sum(((c1 - 1.0) + counts0) * m1h, axis=1, keepdims=True)

    nb = jnp.floor((counts + (BT - 1)) * (1.0 / BT))
    r8 = lax.broadcasted_iota(jnp.int32, (E, E), 0)
    c8 = lax.broadcasted_iota(jnp.int32, (E, E), 1)
    u8 = (r8 <= c8).astype(jnp.float32)
    incl_nb = jnp.dot(nb, u8, preferred_element_type=jnp.float32)
    excl_nb = incl_nb - nb
    po = excl_nb * BT

    p0_ref[...] = (rank0 + jnp.sum(po * m0, axis=1, keepdims=True)
                   ).astype(jnp.int32)
    p1_ref[...] = (rank1 + jnp.sum(po * m1h, axis=1, keepdims=True)
                   ).astype(jnp.int32)

    bcol = lax.broadcasted_iota(jnp.int32, (NBPAD, 1), 0)
    inbi = incl_nb.astype(jnp.int32)
    be = jnp.sum((bcol >= inbi).astype(jnp.int32), axis=1, keepdims=True)
    be_ref[...] = jnp.minimum(be, E - 1)
    nv = jnp.sum(nb).astype(jnp.int32)
    nv_ref[...] = jnp.reshape(nv, (1, 1))


def _router_call(x, router_w, router_b):
    return pl.pallas_call(
        _router_body,
        out_shape=(
            jax.ShapeDtypeStruct((T, 128), jnp.float32),
            jax.ShapeDtypeStruct((T, 128), jnp.float32),
            jax.ShapeDtypeStruct((T, 1), jnp.int32),
            jax.ShapeDtypeStruct((T, 1), jnp.int32),
            jax.ShapeDtypeStruct((NBPAD, 1), jnp.int32),
            jax.ShapeDtypeStruct((1, 1), jnp.int32),
        ),
    )(x, router_w, router_b.reshape(1, E))


def _dispatch_body(x_hbm, p0_hbm, p1_hbm, w0_hbm, w1_hbm, xs_hbm, wsl_hbm,
                   idx0_v, idx1_v, wrow0, wrow1, xbuf,
                   sem0, sem1, sem2, sem3):
    wid = lax.axis_index("s") * 2 + lax.axis_index("c")
    pltpu.sync_copy(p0_hbm.at[wid], idx0_v)
    pltpu.sync_copy(p1_hbm.at[wid], idx1_v)
    for c in range(NCH):
        base = wid * TPW + c * CH
        pltpu.sync_copy(w0_hbm.at[pl.ds(base, CH)], wrow0)
        pltpu.sync_copy(w1_hbm.at[pl.ds(base, CH)], wrow1)
        pltpu.sync_copy(x_hbm.at[pl.ds(base, CH)], xbuf)
        cp0 = pltpu.async_copy(xbuf, xs_hbm.at[idx0_v.at[c]], sem0)
        cp1 = pltpu.async_copy(xbuf, xs_hbm.at[idx1_v.at[c]], sem1)
        cp2 = pltpu.async_copy(wrow0, wsl_hbm.at[idx0_v.at[c]], sem2)
        cp3 = pltpu.async_copy(wrow1, wsl_hbm.at[idx1_v.at[c]], sem3)
        cp0.wait()
        cp1.wait()
        cp2.wait()
        cp3.wait()


def _dispatch_call(x, p0r, p1r, w0r, w1r):
    return pl.kernel(
        _dispatch_body,
        out_type=(jax.ShapeDtypeStruct((P_PAD, D), jnp.float32),
                  jax.ShapeDtypeStruct((P_PAD, 128), jnp.float32)),
        mesh=plsc.VectorSubcoreMesh(core_axis_name="c", subcore_axis_name="s"),
        scratch_types=[
            pltpu.VMEM((NCH, CH), jnp.int32),
            pltpu.VMEM((NCH, CH), jnp.int32),
            pltpu.VMEM((CH, 128), jnp.float32),
            pltpu.VMEM((CH, 128), jnp.float32),
            pltpu.VMEM((CH, D), jnp.float32),
            pltpu.SemaphoreType.DMA,
            pltpu.SemaphoreType.DMA,
            pltpu.SemaphoreType.DMA,
            pltpu.SemaphoreType.DMA,
        ],
    )(x, p0r, p1r, w0r, w1r)


def _ffn_body(be_ref, nv_ref, xs_ref, w1_ref, b1_ref, w2_ref, b2_ref,
              wsl_ref, y_ref, acc_ref, h_ref):
    f = pl.program_id(0)
    b = pl.program_id(1)
    s = pl.program_id(2)

    @pl.when(b < nv_ref[0])
    def _():
        @pl.when(s == 0)
        def _():
            h_ref[...] = jnp.maximum(
                jnp.dot(xs_ref[...], w1_ref[0],
                        preferred_element_type=jnp.float32) + b1_ref[0], 0.0)

        @pl.when(s == 1)
        def _():
            part = jnp.dot(h_ref[...], w2_ref[0],
                           preferred_element_type=jnp.float32)

            @pl.when(f == 0)
            def _():
                acc_ref[b] = part

            @pl.when(jnp.logical_and(f > 0, f < NF - 1))
            def _():
                acc_ref[b] += part

            @pl.when(f == NF - 1)
            def _():
                y_ref[...] = (acc_ref[b] + part + b2_ref[0]) * wsl_ref[:, 0:1]


def _ffn_call(be32, nv1, xs, wsl, w1, b1, w2, b2):
    def eclamp(b, be, nv):
        return be[jnp.where(b < nv[0], b, nv[0] - 1)]

    grid_spec = pltpu.PrefetchScalarGridSpec(
        num_scalar_prefetch=2,
        grid=(NF, NB, 2),
        in_specs=[
            pl.BlockSpec((BT, D),
                         lambda f, b, s, be, nv: (jnp.minimum(b, nv[0] - 1),
                                                  0)),
            pl.BlockSpec((1, D, FB),
                         lambda f, b, s, be, nv: (eclamp(b, be, nv), 0, f)),
            pl.BlockSpec((1, 1, FB),
                         lambda f, b, s, be, nv: (eclamp(b, be, nv), 0, f)),
            pl.BlockSpec((1, FB, D),
                         lambda f, b, s, be, nv: (
                             be[jnp.clip(b - 1 + s, 0, nv[0] - 1)], f, 0)),
            pl.BlockSpec((1, 1, D),
                         lambda f, b, s, be, nv: (eclamp(b, be, nv), 0, 0)),
            pl.BlockSpec((BT, 128),
                         lambda f, b, s, be, nv: (jnp.minimum(b, nv[0] - 1),
                                                  0)),
        ],
        out_specs=pl.BlockSpec(
            (BT, D),
            lambda f, b, s, be, nv: (
                jnp.where(jnp.logical_and(f == NF - 1, s == 1), b, NB), 0)),
        scratch_shapes=[pltpu.VMEM((NB, BT, D), jnp.float32),
                        pltpu.VMEM((BT, FB), jnp.float32)],
    )
    return pl.pallas_call(
        _ffn_body,
        grid_spec=grid_spec,
        out_shape=jax.ShapeDtypeStruct((P_PAD + BT, D), jnp.float32),
        compiler_params=pltpu.CompilerParams(
            dimension_semantics=("arbitrary", "arbitrary", "arbitrary")),
    )(be32, nv1, xs, w1, b1.reshape(E, 1, F), w2, b2.reshape(E, 1, D), wsl)


CHG = 16
NCHG = TPW // CHG


def _gather_body(y_hbm, p0_hbm, p1_hbm, o_hbm, idx0_v, idx1_v,
                 buf0a, buf1a, buf0b, buf1b, sem0a, sem1a, sem0b, sem1b):
    wid = lax.axis_index("s") * 2 + lax.axis_index("c")
    pltpu.sync_copy(p0_hbm.at[wid], idx0_v)
    pltpu.sync_copy(p1_hbm.at[wid], idx1_v)
    bufs = [(buf0a, buf1a, sem0a, sem1a), (buf0b, buf1b, sem0b, sem1b)]

    def start(c):
        b0, b1, s0, s1 = bufs[c % 2]
        cp0 = pltpu.async_copy(y_hbm.at[idx0_v.at[c]], b0, s0)
        cp1 = pltpu.async_copy(y_hbm.at[idx1_v.at[c]], b1, s1)
        return cp0, cp1

    inflight = start(0)
    for c in range(NCHG):
        b0, b1, _, _ = bufs[c % 2]
        cp0, cp1 = inflight
        cp0.wait()
        cp1.wait()
        if c + 1 < NCHG:
            inflight = start(c + 1)

        def row(i, carry):
            for j in range(D // 16):
                sl = pl.ds(j * 16, 16)
                b0[i, sl] = b0[i, sl] + b1[i, sl]
            return carry

        lax.fori_loop(0, CHG, row, 0)
        pltpu.sync_copy(b0, o_hbm.at[pl.ds(wid * TPW + c * CHG, CHG)])


def _gather_call(y, p0g, p1g):
    return pl.kernel(
        _gather_body,
        out_type=jax.ShapeDtypeStruct((T, D), jnp.float32),
        mesh=plsc.VectorSubcoreMesh(core_axis_name="c", subcore_axis_name="s"),
        scratch_types=[
            pltpu.VMEM((NCHG, CHG), jnp.int32),
            pltpu.VMEM((NCHG, CHG), jnp.int32),
            pltpu.VMEM((CHG, D), jnp.float32),
            pltpu.VMEM((CHG, D), jnp.float32),
            pltpu.VMEM((CHG, D), jnp.float32),
            pltpu.VMEM((CHG, D), jnp.float32),
            pltpu.SemaphoreType.DMA,
            pltpu.SemaphoreType.DMA,
            pltpu.SemaphoreType.DMA,
            pltpu.SemaphoreType.DMA,
        ],
    )(y, p0g, p1g)


@jax.jit
def kernel(x, router_w, router_b, w1, b1, w2, b2):
    w0c, w1c, p0, p1, be, nv = _router_call(x, router_w, router_b)
    p0r = p0.reshape(NW, NCH, CH)
    p1r = p1.reshape(NW, NCH, CH)
    xs, wsl = _dispatch_call(x, p0r, p1r, w0c, w1c)
    y = _ffn_call(be.reshape(NBPAD)[:NB], nv.reshape(1), xs, wsl,
                  w1, b1, w2, b2)
    return _gather_call(y, p0.reshape(NW, NCHG, CHG), p1.reshape(NW, NCHG, CHG))


# speedup vs baseline: 1.1231x; 1.1040x over previous
import jax
import jax.numpy as jnp
from jax import lax
from jax.experimental import pallas as pl
from jax.experimental.pallas import tpu as pltpu
from jax.experimental.pallas import tpu_sc as plsc

T, D, E, F, K = 2048, 1024, 8, 4096, 2
BT = 256
FB = 1024
NF = F // FB
NB = 24
NBPAD = 32
P_PAD = NB * BT
NW = 32
TPW = T // NW
CH = 32
NCH = TPW // CH


def _router_body(x_ref, rw_ref, rb_ref, w0_ref, w1_ref, p0_ref, p1_ref,
                 be_ref, nv_ref):
    logits = jnp.dot(x_ref[...], rw_ref[...],
                     preferred_element_type=jnp.float32) + rb_ref[...]
    col = lax.broadcasted_iota(jnp.int32, (T, E), 1)
    m1 = jnp.max(logits, axis=1, keepdims=True)
    i1 = jnp.min(jnp.where(logits == m1, col, E), axis=1, keepdims=True)
    l2 = jnp.where(col == i1, -jnp.inf, logits)
    m2 = jnp.max(l2, axis=1, keepdims=True)
    i2 = jnp.min(jnp.where(l2 == m2, col, E), axis=1, keepdims=True)
    ev = jnp.exp(m2 - m1)
    ones16 = jnp.ones((1, 128), jnp.float32)
    w0_ref[...] = (1.0 / (1.0 + ev)) * ones16
    w1_ref[...] = (ev / (1.0 + ev)) * ones16

    m0 = (col == i1).astype(jnp.float32)
    m1h = (col == i2).astype(jnp.float32)

    def cumsum_t(m):
        c = m
        s = 1
        while s < T:
            c = c + jnp.concatenate(
                [jnp.zeros((s, E), jnp.float32), c[:T - s]], axis=0)
            s *= 2
        return c
    c0 = cumsum_t(m0)
    c1 = cumsum_t(m1h)
    counts0 = c0[T - 1:T, :]
    counts = counts0 + c1[T - 1:T, :]

    rank0 = jnp.sum((c0 - 1.0) * m0, axis=1, keepdims=True)
    rank1 = jnp.sum(((c1 - 1.0) + counts0) * m1h, axis=1, keepdims=True)

    nb = jnp.floor((counts + (BT - 1)) * (1.0 / BT))
    r8 = lax.broadcasted_iota(jnp.int32, (E, E), 0)
    c8 = lax.broadcasted_iota(jnp.int32, (E, E), 1)
    u8 = (r8 <= c8).astype(jnp.float32)
    incl_nb = jnp.dot(nb, u8, preferred_element_type=jnp.float32)
    excl_nb = incl_nb - nb
    po = excl_nb * BT

    p0_ref[...] = (rank0 + jnp.sum(po * m0, axis=1, keepdims=True)
                   ).astype(jnp.int32)
    p1_ref[...] = (rank1 + jnp.sum(po * m1h, axis=1, keepdims=True)
                   ).astype(jnp.int32)

    bcol = lax.broadcasted_iota(jnp.int32, (NBPAD, 1), 0)
    inbi = incl_nb.astype(jnp.int32)
    be = jnp.sum((bcol >= inbi).astype(jnp.int32), axis=1, keepdims=True)
    be_ref[...] = jnp.minimum(be, E - 1)
    nv = jnp.sum(nb).astype(jnp.int32)
    nv_ref[...] = jnp.reshape(nv, (1, 1))


def _router_call(x, router_w, router_b):
    return pl.pallas_call(
        _router_body,
        out_shape=(
            jax.ShapeDtypeStruct((T, 128), jnp.float32),
            jax.ShapeDtypeStruct((T, 128), jnp.float32),
            jax.ShapeDtypeStruct((T, 1), jnp.int32),
            jax.ShapeDtypeStruct((T, 1), jnp.int32),
            jax.ShapeDtypeStruct((NBPAD, 1), jnp.int32),
            jax.ShapeDtypeStruct((1, 1), jnp.int32),
        ),
    )(x, router_w, router_b.reshape(1, E))


def _dispatch_body(x_hbm, p0_hbm, p1_hbm, w0_hbm, w1_hbm, xs_hbm, wsl_hbm,
                   idx0_v, idx1_v, wrow0, wrow1, xbuf,
                   sem0, sem1, sem2, sem3):
    wid = lax.axis_index("s") * 2 + lax.axis_index("c")
    pltpu.sync_copy(p0_hbm.at[wid], idx0_v)
    pltpu.sync_copy(p1_hbm.at[wid], idx1_v)
    for c in range(NCH):
        base = wid * TPW + c * CH
        pltpu.sync_copy(w0_hbm.at[pl.ds(base, CH)], wrow0)
        pltpu.sync_copy(w1_hbm.at[pl.ds(base, CH)], wrow1)
        pltpu.sync_copy(x_hbm.at[pl.ds(base, CH)], xbuf)
        cp0 = pltpu.async_copy(xbuf, xs_hbm.at[idx0_v.at[c]], sem0)
        cp1 = pltpu.async_copy(xbuf, xs_hbm.at[idx1_v.at[c]], sem1)
        cp2 = pltpu.async_copy(wrow0, wsl_hbm.at[idx0_v.at[c]], sem2)
        cp3 = pltpu.async_copy(wrow1, wsl_hbm.at[idx1_v.at[c]], sem3)
        cp0.wait()
        cp1.wait()
        cp2.wait()
        cp3.wait()


def _dispatch_call(x, p0r, p1r, w0r, w1r):
    return pl.kernel(
        _dispatch_body,
        out_type=(jax.ShapeDtypeStruct((P_PAD, D), jnp.float32),
                  jax.ShapeDtypeStruct((P_PAD, 128), jnp.float32)),
        mesh=plsc.VectorSubcoreMesh(core_axis_name="c", subcore_axis_name="s"),
        scratch_types=[
            pltpu.VMEM((NCH, CH), jnp.int32),
            pltpu.VMEM((NCH, CH), jnp.int32),
            pltpu.VMEM((CH, 128), jnp.float32),
            pltpu.VMEM((CH, 128), jnp.float32),
            pltpu.VMEM((CH, D), jnp.float32),
            pltpu.SemaphoreType.DMA,
            pltpu.SemaphoreType.DMA,
            pltpu.SemaphoreType.DMA,
            pltpu.SemaphoreType.DMA,
        ],
    )(x, p0r, p1r, w0r, w1r)


def _ffn_body(be_ref, nv_ref, xs_ref, w1_ref, b1_ref, w2_ref, b2_ref,
              wsl_ref, y_ref, acc_ref):
    f = pl.program_id(0)
    b = pl.program_id(1)

    @pl.when(b < nv_ref[0])
    def _():
        h = jnp.maximum(
            jnp.dot(xs_ref[...], w1_ref[0],
                    preferred_element_type=jnp.float32) + b1_ref[0], 0.0)
        part = jnp.dot(h, w2_ref[0], preferred_element_type=jnp.float32)

        @pl.when(f == 0)
        def _():
            acc_ref[b] = part

        @pl.when(jnp.logical_and(f > 0, f < NF - 1))
        def _():
            acc_ref[b] += part

        @pl.when(f == NF - 1)
        def _():
            y_ref[...] = (acc_ref[b] + part + b2_ref[0]) * wsl_ref[:, 0:1]


def _ffn_call(be32, nv1, xs, wsl, w1, b1, w2, b2):
    def eclamp(f, b, be, nv):
        return be[jnp.where(b < nv[0], b, nv[0] - 1)]

    grid_spec = pltpu.PrefetchScalarGridSpec(
        num_scalar_prefetch=2,
        grid=(NF, NB),
        in_specs=[
            pl.BlockSpec((BT, D),
                         lambda f, b, be, nv: (jnp.minimum(b, nv[0] - 1), 0)),
            pl.BlockSpec((1, D, FB),
                         lambda f, b, be, nv: (eclamp(f, b, be, nv), 0, f)),
            pl.BlockSpec((1, 1, FB),
                         lambda f, b, be, nv: (eclamp(f, b, be, nv), 0, f)),
            pl.BlockSpec((1, FB, D),
                         lambda f, b, be, nv: (eclamp(f, b, be, nv), f, 0)),
            pl.BlockSpec((1, 1, D),
                         lambda f, b, be, nv: (eclamp(f, b, be, nv), 0, 0)),
            pl.BlockSpec((BT, 128),
                         lambda f, b, be, nv: (jnp.minimum(b, nv[0] - 1), 0)),
        ],
        out_specs=pl.BlockSpec(
            (BT, D), lambda f, b, be, nv: (jnp.where(f == NF - 1, b, NB), 0)),
        scratch_shapes=[pltpu.VMEM((NB, BT, D), jnp.float32)],
    )
    return pl.pallas_call(
        _ffn_body,
        grid_spec=grid_spec,
        out_shape=jax.ShapeDtypeStruct((P_PAD + BT, D), jnp.float32),
        compiler_params=pltpu.CompilerParams(
            dimension_semantics=("arbitrary", "arbitrary")),
    )(be32, nv1, xs, w1, b1.reshape(E, 1, F), w2, b2.reshape(E, 1, D), wsl)


CHG = 16
NCHG = TPW // CHG


def _gather_body(y_hbm, p0_hbm, p1_hbm, o_hbm, idx0_v, idx1_v,
                 buf0a, buf1a, buf0b, buf1b, sem0a, sem1a, sem0b, sem1b):
    wid = lax.axis_index("s") * 2 + lax.axis_index("c")
    pltpu.sync_copy(p0_hbm.at[wid], idx0_v)
    pltpu.sync_copy(p1_hbm.at[wid], idx1_v)
    bufs = [(buf0a, buf1a, sem0a, sem1a), (buf0b, buf1b, sem0b, sem1b)]

    def start(c):
        b0, b1, s0, s1 = bufs[c % 2]
        cp0 = pltpu.async_copy(y_hbm.at[idx0_v.at[c]], b0, s0)
        cp1 = pltpu.async_copy(y_hbm.at[idx1_v.at[c]], b1, s1)
        return cp0, cp1

    inflight = start(0)
    for c in range(NCHG):
        b0, b1, _, _ = bufs[c % 2]
        cp0, cp1 = inflight
        cp0.wait()
        cp1.wait()
        if c + 1 < NCHG:
            inflight = start(c + 1)

        def row(i, carry):
            for j in range(D // 16):
                sl = pl.ds(j * 16, 16)
                b0[i, sl] = b0[i, sl] + b1[i, sl]
            return carry

        lax.fori_loop(0, CHG, row, 0)
        pltpu.sync_copy(b0, o_hbm.at[pl.ds(wid * TPW + c * CHG, CHG)])


def _gather_call(y, p0g, p1g):
    return pl.kernel(
        _gather_body,
        out_type=jax.ShapeDtypeStruct((T, D), jnp.float32),
        mesh=plsc.VectorSubcoreMesh(core_axis_name="c", subcore_axis_name="s"),
        scratch_types=[
            pltpu.VMEM((NCHG, CHG), jnp.int32),
            pltpu.VMEM((NCHG, CHG), jnp.int32),
            pltpu.VMEM((CHG, D), jnp.float32),
            pltpu.VMEM((CHG, D), jnp.float32),
            pltpu.VMEM((CHG, D), jnp.float32),
            pltpu.VMEM((CHG, D), jnp.float32),
            pltpu.SemaphoreType.DMA,
            pltpu.SemaphoreType.DMA,
            pltpu.SemaphoreType.DMA,
            pltpu.SemaphoreType.DMA,
        ],
    )(y, p0g, p1g)


@jax.jit
def kernel(x, router_w, router_b, w1, b1, w2, b2):
    w0c, w1c, p0, p1, be, nv = _router_call(x, router_w, router_b)
    p0r = p0.reshape(NW, NCH, CH)
    p1r = p1.reshape(NW, NCH, CH)
    xs, wsl = _dispatch_call(x, p0r, p1r, w0c, w1c)
    y = _ffn_call(be.reshape(NBPAD)[:NB], nv.reshape(1), xs, wsl,
                  w1, b1, w2, b2)
    return _gather_call(y, p0.reshape(NW, NCHG, CHG), p1.reshape(NW, NCHG, CHG))
